# baseline (speedup 1.0000x reference)
import jax
import jax.numpy as jnp
from jax.experimental import pallas as pl
from jax.experimental.pallas import tpu as pltpu

_K = 32
_QB = 256
_WS = 4608
_WPAD = 12800


def _knn_block(q_ref, r_ref, o_ref, d_ref):
    q = q_ref[...]
    qb = q[:, 0:1]
    qx = q[:, 1:2]
    qy = q[:, 2:3]
    qz = q[:, 3:4]

    b_lo = jnp.min(qb)
    rb_full = r_ref[0:1, :]
    r_lo = jnp.sum((rb_full < b_lo).astype(jnp.int32))
    s0 = (r_lo // 128) * 128

    rb = r_ref[0:1, pl.ds(s0, _WS)]
    rx = r_ref[1:2, pl.ds(s0, _WS)]
    ry = r_ref[2:3, pl.ds(s0, _WS)]
    rz = r_ref[3:4, pl.ds(s0, _WS)]

    q2 = qx * qx + qy * qy + qz * qz
    r2 = rx * rx + ry * ry + rz * rz
    bf = jnp.bfloat16
    f32 = jnp.float32
    qxb = qx.astype(bf).astype(f32)
    qyb = qy.astype(bf).astype(f32)
    qzb = qz.astype(bf).astype(f32)
    rxb = rx.astype(bf).astype(f32)
    ryb = ry.astype(bf).astype(f32)
    rzb = rz.astype(bf).astype(f32)
    qr = qxb * rxb + qyb * ryb + qzb * rzb
    dist = (q2 + r2) - 2.0 * qr
    dist = jnp.where(qb != rb, jnp.float32(1e30), dist)
    d_ref[...] = dist

    iota = jax.lax.broadcasted_iota(jnp.int32, (1, _WS), 1)
    lane = jax.lax.broadcasted_iota(jnp.int32, (1, _K), 1)

    def body(k, best):
        dmat = d_ref[...]
        isel = jnp.argmin(dmat, axis=1, keepdims=True).astype(jnp.int32)
        d_ref[...] = jnp.where(iota == isel, jnp.float32(jnp.inf), dmat)
        return jnp.where(lane == k, isel, best)

    best = jax.lax.fori_loop(
        0, _K, body, jnp.zeros((_QB, _K), jnp.int32))
    o_ref[...] = best + s0


def kernel(ref_bxyz, query_bxyz):
    m = query_bxyz.shape[0]
    n = ref_bxyz.shape[0]
    rt = jnp.transpose(ref_bxyz)
    rt = jnp.concatenate(
        [rt, jnp.full((4, _WPAD - n), 1e9, jnp.float32)], axis=1)
    rt = jnp.concatenate(
        [rt, jnp.zeros((4, _WPAD), jnp.float32)], axis=0)

    out = pl.pallas_call(
        _knn_block,
        grid=(m // _QB,),
        in_specs=[
            pl.BlockSpec((_QB, 4), lambda i: (i, 0)),
            pl.BlockSpec((8, _WPAD), lambda i: (0, 0)),
        ],
        out_specs=pl.BlockSpec((_QB, _K), lambda i: (i, 0)),
        out_shape=jax.ShapeDtypeStruct((m, _K), jnp.int32),
        scratch_shapes=[pltpu.VMEM((_QB, _WS), jnp.float32)],
    )(query_bxyz, rt)

    e_ref = out.reshape(-1)
    e_query = jnp.broadcast_to(
        jnp.arange(m, dtype=jnp.int32)[:, None], (m, _K)).reshape(-1)
    return (e_ref, e_query)


# speedup vs baseline: 1.6888x; 1.6888x over previous
import jax
import jax.numpy as jnp
from jax.experimental import pallas as pl
from jax.experimental.pallas import tpu as pltpu

_K = 32
_QB = 256
_WN = 2560
_WS = 4608
_WPAD = 12800


def _process(width, s0, q, q_parts, r_ref, o_ref, d_ref):
    qb, qx, qy, qz = q_parts
    rb = r_ref[0:1, pl.ds(s0, width)]
    rx = r_ref[1:2, pl.ds(s0, width)]
    ry = r_ref[2:3, pl.ds(s0, width)]
    rz = r_ref[3:4, pl.ds(s0, width)]

    q2 = qx * qx + qy * qy + qz * qz
    r2 = rx * rx + ry * ry + rz * rz
    bf = jnp.bfloat16
    f32 = jnp.float32
    qxb = qx.astype(bf).astype(f32)
    qyb = qy.astype(bf).astype(f32)
    qzb = qz.astype(bf).astype(f32)
    rxb = rx.astype(bf).astype(f32)
    ryb = ry.astype(bf).astype(f32)
    rzb = rz.astype(bf).astype(f32)
    qr = qxb * rxb + qyb * ryb + qzb * rzb
    dist = (q2 + r2) - 2.0 * qr
    dist = jnp.where(qb != rb, jnp.float32(1e30), dist)
    d_ref[:, 0:width] = dist

    iota = jax.lax.broadcasted_iota(jnp.int32, (1, width), 1)
    lane = jax.lax.broadcasted_iota(jnp.int32, (1, _K), 1)

    def body(k, best):
        dmat = d_ref[:, 0:width]
        m = jnp.min(dmat, axis=1, keepdims=True)
        isel = jnp.min(
            jnp.where(dmat == m, iota, jnp.int32(2**31 - 1)),
            axis=1, keepdims=True)
        d_ref[:, 0:width] = jnp.where(
            iota == isel, jnp.float32(jnp.inf), dmat)
        return jnp.where(lane == k, isel, best)

    best = jax.lax.fori_loop(
        0, _K, body, jnp.zeros((_QB, _K), jnp.int32))
    o_ref[...] = best + s0


def _knn_block(q_ref, r_ref, o_ref, d_ref):
    q = q_ref[...]
    qb = q[:, 0:1]
    qx = q[:, 1:2]
    qy = q[:, 2:3]
    qz = q[:, 3:4]

    b_lo = jnp.min(qb)
    b_hi = jnp.max(qb)
    rb_full = r_ref[0:1, :]
    r_lo = jnp.sum((rb_full < b_lo).astype(jnp.int32))
    r_hi = jnp.sum((rb_full <= b_hi).astype(jnp.int32))
    s0 = (r_lo // 128) * 128
    fits = (r_hi - s0) <= _WN

    @pl.when(fits)
    def _narrow():
        _process(_WN, s0, q, (qb, qx, qy, qz), r_ref, o_ref, d_ref)

    @pl.when(jnp.logical_not(fits))
    def _wide():
        _process(_WS, s0, q, (qb, qx, qy, qz), r_ref, o_ref, d_ref)


def kernel(ref_bxyz, query_bxyz):
    m = query_bxyz.shape[0]
    n = ref_bxyz.shape[0]
    rt = jnp.transpose(ref_bxyz)
    rt = jnp.concatenate(
        [rt, jnp.full((4, _WPAD - n), 1e9, jnp.float32)], axis=1)
    rt = jnp.concatenate(
        [rt, jnp.zeros((4, _WPAD), jnp.float32)], axis=0)

    out = pl.pallas_call(
        _knn_block,
        grid=(m // _QB,),
        in_specs=[
            pl.BlockSpec((_QB, 4), lambda i: (i, 0)),
            pl.BlockSpec((8, _WPAD), lambda i: (0, 0)),
        ],
        out_specs=pl.BlockSpec((_QB, _K), lambda i: (i, 0)),
        out_shape=jax.ShapeDtypeStruct((m, _K), jnp.int32),
        scratch_shapes=[pltpu.VMEM((_QB, _WS), jnp.float32)],
    )(query_bxyz, rt)

    e_ref = out.reshape(-1)
    e_query = jnp.broadcast_to(
        jnp.arange(m, dtype=jnp.int32)[:, None], (m, _K)).reshape(-1)
    return (e_ref, e_query)


# speedup vs baseline: 1.7190x; 1.0179x over previous
import jax
import jax.numpy as jnp
from jax.experimental import pallas as pl
from jax.experimental.pallas import tpu as pltpu

_K = 32
_QB = 256
_WN = 2560
_WS = 4608
_WPAD = 12800


def _process(width, s0, q, q_parts, r_ref, o_ref, d_ref):
    qb, qx, qy, qz = q_parts
    rb = r_ref[0:1, pl.ds(s0, width)]
    rx = r_ref[1:2, pl.ds(s0, width)]
    ry = r_ref[2:3, pl.ds(s0, width)]
    rz = r_ref[3:4, pl.ds(s0, width)]

    q2 = qx * qx + qy * qy + qz * qz
    r2 = rx * rx + ry * ry + rz * rz
    bf = jnp.bfloat16
    f32 = jnp.float32
    qmat = jnp.concatenate([qx, qy, qz], axis=1).astype(bf)
    rmat = jnp.concatenate([rx, ry, rz], axis=0).astype(bf)
    qr = jnp.dot(qmat, rmat, preferred_element_type=f32)
    dist = (q2 + r2) - 2.0 * qr
    dist = jnp.where(qb != rb, jnp.float32(1e30), dist)
    d_ref[:, 0:width] = dist

    iota = jax.lax.broadcasted_iota(jnp.int32, (1, width), 1)
    lane = jax.lax.broadcasted_iota(jnp.int32, (1, _K), 1)

    def body(k, best):
        dmat = d_ref[:, 0:width]
        m = jnp.min(dmat, axis=1, keepdims=True)
        isel = jnp.min(
            jnp.where(dmat == m, iota, jnp.int32(2**31 - 1)),
            axis=1, keepdims=True)
        d_ref[:, 0:width] = jnp.where(
            iota == isel, jnp.float32(jnp.inf), dmat)
        return jnp.where(lane == k, isel, best)

    best = jax.lax.fori_loop(
        0, _K, body, jnp.zeros((_QB, _K), jnp.int32))
    o_ref[...] = best + s0


def _knn_block(q_ref, r_ref, o_ref, d_ref):
    q = q_ref[...]
    qb = q[:, 0:1]
    qx = q[:, 1:2]
    qy = q[:, 2:3]
    qz = q[:, 3:4]

    b_lo = jnp.min(qb)
    b_hi = jnp.max(qb)
    rb_full = r_ref[0:1, :]
    r_lo = jnp.sum((rb_full < b_lo).astype(jnp.int32))
    r_hi = jnp.sum((rb_full <= b_hi).astype(jnp.int32))
    s0 = (r_lo // 128) * 128
    fits = (r_hi - s0) <= _WN

    @pl.when(fits)
    def _narrow():
        _process(_WN, s0, q, (qb, qx, qy, qz), r_ref, o_ref, d_ref)

    @pl.when(jnp.logical_not(fits))
    def _wide():
        _process(_WS, s0, q, (qb, qx, qy, qz), r_ref, o_ref, d_ref)


def kernel(ref_bxyz, query_bxyz):
    m = query_bxyz.shape[0]
    n = ref_bxyz.shape[0]
    rt = jnp.transpose(ref_bxyz)
    rt = jnp.concatenate(
        [rt, jnp.full((4, _WPAD - n), 1e9, jnp.float32)], axis=1)
    rt = jnp.concatenate(
        [rt, jnp.zeros((4, _WPAD), jnp.float32)], axis=0)

    out = pl.pallas_call(
        _knn_block,
        grid=(m // _QB,),
        in_specs=[
            pl.BlockSpec((_QB, 4), lambda i: (i, 0)),
            pl.BlockSpec((8, _WPAD), lambda i: (0, 0)),
        ],
        out_specs=pl.BlockSpec((_QB, _K), lambda i: (i, 0)),
        out_shape=jax.ShapeDtypeStruct((m, _K), jnp.int32),
        scratch_shapes=[pltpu.VMEM((_QB, _WS), jnp.float32)],
    )(query_bxyz, rt)

    e_ref = out.reshape(-1)
    e_query = jnp.broadcast_to(
        jnp.arange(m, dtype=jnp.int32)[:, None], (m, _K)).reshape(-1)
    return (e_ref, e_query)
